# baseline (speedup 1.0000x reference)
import functools

import jax
import jax.numpy as jnp
from jax import lax
from jax.experimental import pallas as pl
from jax.experimental.pallas import tpu as pltpu
from jax.experimental.pallas import tpu_sc as plsc

N_NODES = 10000
N_EDGES = 320000
NB = 32
NPB = 320
NPAD = NB * NPB
K = 64
CE = 2000
SB = 4224
FLUSH = 2048
RB = 8192
CPB = RB // K
CAP = 40 * RB
DSHIFT = 9
DMASK = 511
DUMP = NPB
NEG_INF = float("-inf")

_sc_mesh = plsc.VectorSubcoreMesh(
    core_axis_name="c", subcore_axis_name="s", num_cores=2, num_subcores=16)
_sc_params = pltpu.CompilerParams(needs_layout_passes=False)


def _wid():
    return lax.axis_index("s") * 2 + lax.axis_index("c")


@functools.partial(
    pl.kernel,
    out_type=(
        jax.ShapeDtypeStruct((NB * CAP,), jnp.int32),
        jax.ShapeDtypeStruct((NB * 16,), jnp.int32),
    ),
    mesh=_sc_mesh,
    compiler_params=_sc_params,
    scratch_types=[
        pltpu.VMEM((CE,), jnp.int32),
        pltpu.VMEM((CE,), jnp.int32),
        pltpu.VMEM((CE,), jnp.int32),
        pltpu.VMEM((CE,), jnp.int32),
        pltpu.VMEM((SB,), jnp.int32),
        pltpu.VMEM((16,), jnp.int32),
        pltpu.SemaphoreType.DMA,
        pltpu.SemaphoreType.DMA,
    ],
)
def _bucket(src_hbm, dst_hbm, recs_hbm, cnt_hbm,
            src0, dst0, src1, dst1, staging, cnt_v, sem0, sem1):
    b = _wid()
    lo = b * NPB
    hi = lo + NPB
    srcs = (src0, src1)
    dsts = (dst0, dst1)
    sems = (sem0, sem1)
    NCH = N_EDGES // CE

    def _issue(c, p):
        pltpu.async_copy(src_hbm.at[pl.ds(c * CE, CE)], srcs[p], sems[p])
        pltpu.async_copy(dst_hbm.at[pl.ds(c * CE, CE)], dsts[p], sems[p])

    def _wait(p):
        pltpu.make_async_copy(src_hbm.at[pl.ds(0, CE)], srcs[p],
                              sems[p]).wait()
        pltpu.make_async_copy(dst_hbm.at[pl.ds(0, CE)], dsts[p],
                              sems[p]).wait()

    def _compress(p, off):
        def _grp(g, off):
            s16 = srcs[p][pl.ds(g * 16, 16)]
            d16 = dsts[p][pl.ds(g * 16, 16)]
            m = jnp.logical_and(d16 >= lo, d16 < hi)
            rec = jnp.bitwise_or(lax.shift_left(s16, DSHIFT), d16 - lo)
            cs = plsc.cumsum(m.astype(jnp.int32))
            offv = jnp.full((16,), off, jnp.int32)
            plsc.store_scatter(staging, [offv + cs - 1], rec, mask=m)
            return off + cs[15]
        return lax.fori_loop(0, CE // 16, _grp, off)

    def _maybe_flush(carry):
        off, gofs = carry

        def _flush(c):
            off, gofs = c
            pltpu.sync_copy(
                staging.at[pl.ds(0, FLUSH)],
                recs_hbm.at[pl.ds(pl.multiple_of(b * CAP + gofs, 16), FLUSH)])
            nrem = off - FLUSH

            def _mv(i, _):
                staging[pl.ds(i * 16, 16)] = \
                    staging[pl.ds(FLUSH + i * 16, 16)]
                return 0
            lax.fori_loop(0, lax.div(nrem + 15, jnp.int32(16)), _mv, 0)
            return off - FLUSH, gofs + FLUSH

        return lax.cond(off >= FLUSH, _flush, lambda c: c, (off, gofs))

    _issue(0, 0)

    def _pair2(t, carry):
        c = 2 * t

        @pl.when(c + 1 < NCH)
        def _():
            _issue(c + 1, 1)

        _wait(0)
        carry = _maybe_flush((_compress(0, carry[0]), carry[1]))

        @pl.when(c + 2 < NCH)
        def _():
            _issue(c + 2, 0)

        _wait(1)
        carry = _maybe_flush((_compress(1, carry[0]), carry[1]))
        return carry

    off, gofs = lax.fori_loop(0, NCH // 2, _pair2,
                              (jnp.int32(0), jnp.int32(0)))

    dumpv = jnp.full((16,), DUMP, jnp.int32)
    staging[pl.ds(off, 16)] = dumpv
    staging[pl.ds(off + 16, 16)] = dumpv
    staging[pl.ds(off + 32, 16)] = dumpv
    staging[pl.ds(off + 48, 16)] = dumpv
    pltpu.sync_copy(
        staging, recs_hbm.at[pl.ds(pl.multiple_of(b * CAP + gofs, 16), SB)])
    cnt_v[...] = jnp.full((16,), gofs + off, jnp.int32)
    pltpu.sync_copy(cnt_v,
                    cnt_hbm.at[pl.ds(pl.multiple_of(b * 16, 16), 16)])


def _make_agg(d_feat):
    kv = d_feat // 16

    @functools.partial(
        pl.kernel,
        out_type=jax.ShapeDtypeStruct((NPAD * d_feat,), jnp.float32),
        mesh=_sc_mesh,
        compiler_params=_sc_params,
        scratch_types=[
            pltpu.VMEM(((NPB + 1) * d_feat,), jnp.float32),
            pltpu.VMEM((K, d_feat), jnp.float32),
            pltpu.VMEM((K, d_feat), jnp.float32),
            pltpu.VMEM((RB,), jnp.int32),
            pltpu.VMEM((K,), jnp.int32),
            pltpu.VMEM((K,), jnp.int32),
            pltpu.VMEM((16,), jnp.int32),
            pltpu.SemaphoreType.DMA,
            pltpu.SemaphoreType.DMA,
        ],
    )
    def _agg(x_hbm, recs_hbm, cnt_hbm, out_hbm,
             acc, rows0, rows1, rec_b, idx0, idx1, cnt_v, sem0, sem1):
        b = _wid()
        ninf = jnp.full((16,), NEG_INF, jnp.float32)
        rowss = (rows0, rows1)
        idxs = (idx0, idx1)
        sems = (sem0, sem1)

        def _init(i, _):
            acc[pl.ds(i * 16, 16)] = ninf
            return 0
        lax.fori_loop(0, (NPB + 1) * kv, _init, 0)

        pltpu.sync_copy(cnt_hbm.at[pl.ds(pl.multiple_of(b * 16, 16), 16)],
                        cnt_v)
        cntb = cnt_v[...][0]
        nch = lax.div(cntb + (K - 1), jnp.int32(K))

        def _issue(ch, p):
            for g in range(K // 16):
                idxs[p][pl.ds(g * 16, 16)] = lax.shift_right_logical(
                    rec_b[pl.ds(pl.multiple_of(ch * K + g * 16, 16), 16)],
                    DSHIFT)
            pltpu.async_copy(x_hbm.at[idxs[p]], rowss[p], sems[p])

        lane_consts = [jnp.full((16,), l, jnp.int32) for l in range(16)]
        col_consts = [lax.iota(jnp.int32, 16) + k * 16 for k in range(kv)]

        def _finish(ch, p):
            pltpu.make_async_copy(x_hbm.at[idxs[p]], rowss[p], sems[p]).wait()

            def _grp(g, _):
                grp = rec_b[pl.ds(pl.multiple_of(ch * K + g * 16, 16), 16)]
                dgrp = jnp.bitwise_and(grp, DMASK)
                for l in range(16):
                    db = lax.gather(
                        dgrp, lane_consts[l].reshape(16, 1),
                        lax.GatherDimensionNumbers(
                            offset_dims=(), collapsed_slice_dims=(0,),
                            start_index_map=(0,)),
                        (1,), mode=lax.GatherScatterMode.PROMISE_IN_BOUNDS)
                    base = db * jnp.int32(d_feat)
                    e = g * 16 + l
                    for k in range(kv):
                        a_idx = base + col_consts[k]
                        cur = plsc.load_gather(acc, [a_idx])
                        r = rowss[p][e, pl.ds(k * 16, 16)]
                        plsc.store_scatter(acc, [a_idx],
                                           jnp.maximum(cur, r))
                return 0
            lax.fori_loop(0, K // 16, _grp, 0)

        nblk = lax.div(cntb + (RB - 1), jnp.int32(RB))

        def _blk(blk, _):
            pltpu.sync_copy(
                recs_hbm.at[pl.ds(pl.multiple_of(b * CAP + blk * RB, 64), RB)],
                rec_b)
            nch_here = jnp.minimum(jnp.int32(CPB), nch - blk * CPB)

            @pl.when(nch_here > 0)
            def _():
                _issue(0, 0)

            def _pair(t, _):
                c0 = 2 * t
                c1 = c0 + 1
                c2 = c0 + 2

                @pl.when(c1 < nch_here)
                def _():
                    _issue(c1, 1)

                _finish(c0, 0)

                @pl.when(c2 < nch_here)
                def _():
                    _issue(c2, 0)

                @pl.when(c1 < nch_here)
                def _():
                    _finish(c1, 1)
                return 0
            lax.fori_loop(0, lax.div(nch_here + 1, jnp.int32(2)), _pair, 0)
            return 0
        lax.fori_loop(0, nblk, _blk, 0)

        def _fix(i, _):
            v = acc[pl.ds(i * 16, 16)]
            acc[pl.ds(i * 16, 16)] = jnp.where(v == NEG_INF, 0.0, v)
            return 0
        lax.fori_loop(0, NPB * kv, _fix, 0)
        pltpu.sync_copy(
            acc.at[pl.ds(0, NPB * d_feat)],
            out_hbm.at[pl.ds(pl.multiple_of(b * NPB * d_feat, 8),
                             NPB * d_feat)])

    return _agg


_agg128 = _make_agg(128)
_agg256 = _make_agg(256)


BN = 1000
_GRID = N_NODES // BN


def _row_spec(d):
    return pl.BlockSpec((BN, d), lambda i: (i, 0))


def _full_spec(shape):
    nd = len(shape)
    return pl.BlockSpec(shape, lambda i: (0,) * nd)


def _tc_call(fn, ins, out_widths):
    in_specs = []
    for a in ins:
        if a.shape[0] == N_NODES:
            in_specs.append(_row_spec(a.shape[1]))
        else:
            in_specs.append(_full_spec(a.shape))
    multi = len(out_widths) > 1
    out_shape = tuple(jax.ShapeDtypeStruct((N_NODES, w), jnp.float32)
                      for w in out_widths)
    out_specs = tuple(_row_spec(w) for w in out_widths)
    if not multi:
        out_shape, out_specs = out_shape[0], out_specs[0]
    return pl.pallas_call(
        fn, grid=(_GRID,), in_specs=in_specs,
        out_specs=out_specs, out_shape=out_shape)(*ins)


def _stage_a(agg_ref, x_ref, wl, bl, wr, h_ref):
    h_ref[...] = jax.nn.relu(
        jnp.dot(agg_ref[...], wl[...], preferred_element_type=jnp.float32)
        + bl[...]
        + jnp.dot(x_ref[...], wr[...], preferred_element_type=jnp.float32))


def _stage_b(agg_ref, h_ref, wl, bl, wr, u1_ref, u2_ref):
    t = jax.nn.relu(
        jnp.dot(agg_ref[...], wl[...], preferred_element_type=jnp.float32)
        + bl[...]
        + jnp.dot(h_ref[...], wr[...], preferred_element_type=jnp.float32))
    u1_ref[...] = t[:, :256]
    u2_ref[...] = t[:, 256:]


def _stage_c(a1_ref, a2_ref, u1_ref, u2_ref, wl, bl, wr, v_ref, rm_ref):
    agg = jnp.concatenate([a1_ref[...], a2_ref[...]], axis=1)
    u = jnp.concatenate([u1_ref[...], u2_ref[...]], axis=1)
    t = jax.nn.relu(
        jnp.dot(agg, wl[...], preferred_element_type=jnp.float32)
        + bl[...]
        + jnp.dot(u, wr[...], preferred_element_type=jnp.float32))
    v_ref[...] = t
    rm_ref[...] = jnp.concatenate(
        [t[:, :64], jnp.zeros((BN, 64), jnp.float32)], axis=1)


def _stage_d(aggrm_ref, v_ref, wl3, bl3, wr3, wrt, brt, wmd, bmd, wj, bj,
             o_ref):
    v = v_ref[...]
    rm_pre = (jnp.dot(aggrm_ref[...][:, :64], wl3[...],
                      preferred_element_type=jnp.float32)
              + bl3[...]
              + jnp.dot(v[:, :64], wr3[...],
                        preferred_element_type=jnp.float32))
    m = jnp.max(rm_pre, axis=1, keepdims=True)
    ex = jnp.exp(rm_pre - m)
    ls = rm_pre - m - jnp.log(jnp.sum(ex, axis=1, keepdims=True))
    rt = jnp.dot(v[:, 64:128], wrt[...],
                 preferred_element_type=jnp.float32) + brt[...]
    md = jnp.dot(v[:, 128:192], wmd[...],
                 preferred_element_type=jnp.float32) + bmd[...]
    jj = jnp.dot(v[:, 192:256], wj[...],
                 preferred_element_type=jnp.float32) + bj[...]
    o_ref[...] = jnp.concatenate(
        [ls, rt, md, jj, jnp.zeros((BN, 2), jnp.float32)], axis=1)


def _block_diag4(ws):
    out = jnp.zeros((512, 256), jnp.float32)
    for i, w in enumerate(ws):
        out = out.at[i * 128:(i + 1) * 128, i * 64:(i + 1) * 64].set(w)
    return out


def kernel(x, edge_index, params):
    p = params
    src = edge_index[0]
    dst = edge_index[1]
    recs, cnt = _bucket(src, dst)

    aggx = _agg128(x, recs, cnt).reshape(NPAD, 128)[:N_NODES]
    h = _tc_call(_stage_a,
                 (aggx, x, p["shared"]["Wl"], p["shared"]["bl"].reshape(1, -1),
                  p["shared"]["Wr"]), (256,))

    aggh = _agg256(h, recs, cnt).reshape(NPAD, 256)[:N_NODES]
    wl_cat = jnp.concatenate(
        [p[k]["Wl"] for k in ("rm1", "rt1", "md1", "j1")], axis=1)
    bl_cat = jnp.concatenate(
        [p[k]["bl"] for k in ("rm1", "rt1", "md1", "j1")]).reshape(1, -1)
    wr_cat = jnp.concatenate(
        [p[k]["Wr"] for k in ("rm1", "rt1", "md1", "j1")], axis=1)
    u1, u2 = _tc_call(_stage_b, (aggh, h, wl_cat, bl_cat, wr_cat), (256, 256))

    a1 = _agg256(u1, recs, cnt).reshape(NPAD, 256)[:N_NODES]
    a2 = _agg256(u2, recs, cnt).reshape(NPAD, 256)[:N_NODES]
    wl_bd = _block_diag4([p[k]["Wl"] for k in ("rm2", "rt2", "md2", "j2")])
    bl_bd = jnp.concatenate(
        [p[k]["bl"] for k in ("rm2", "rt2", "md2", "j2")]).reshape(1, -1)
    wr_bd = _block_diag4([p[k]["Wr"] for k in ("rm2", "rt2", "md2", "j2")])
    v, rm2 = _tc_call(_stage_c, (a1, a2, u1, u2, wl_bd, bl_bd, wr_bd),
                      (256, 128))

    aggrm = _agg128(rm2, recs, cnt).reshape(NPAD, 128)[:N_NODES]
    o = _tc_call(_stage_d,
                 (aggrm, v, p["rm3"]["Wl"], p["rm3"]["bl"].reshape(1, -1),
                  p["rm3"]["Wr"], p["rt3"]["W"], p["rt3"]["b"].reshape(1, -1),
                  p["md3"]["W"], p["md3"]["b"].reshape(1, -1),
                  p["j3"]["W"], p["j3"]["b"].reshape(1, -1)), (8,))

    rm = o[:, 0:2]
    rt = o[:, 2]
    md = o[:, 3]
    j = o[:, 4:6].reshape(-1)
    return (rm, rt, md, j)


# speedup vs baseline: 2.8921x; 2.8921x over previous
import functools

import jax
import jax.numpy as jnp
from jax import lax
from jax.experimental import pallas as pl
from jax.experimental.pallas import tpu as pltpu
from jax.experimental.pallas import tpu_sc as plsc

N_NODES = 10000
N_EDGES = 320000
NB = 32
NPB = 320
NPAD = NB * NPB
K = 64
CE = 2000
SB = 4224
FLUSH = 2048
RB = 8192
CPB = RB // K
CAP = 40 * RB
DSHIFT = 9
DMASK = 511
DUMP = NPB
NEG_INF = float("-inf")

_sc_mesh = plsc.VectorSubcoreMesh(
    core_axis_name="c", subcore_axis_name="s", num_cores=2, num_subcores=16)
_sc_params = pltpu.CompilerParams(needs_layout_passes=False)


def _wid():
    return lax.axis_index("s") * 2 + lax.axis_index("c")


@functools.partial(
    pl.kernel,
    out_type=(
        jax.ShapeDtypeStruct((NB * CAP,), jnp.int32),
        jax.ShapeDtypeStruct((NB * 16,), jnp.int32),
    ),
    mesh=_sc_mesh,
    compiler_params=_sc_params,
    scratch_types=[
        pltpu.VMEM((CE,), jnp.int32),
        pltpu.VMEM((CE,), jnp.int32),
        pltpu.VMEM((CE,), jnp.int32),
        pltpu.VMEM((CE,), jnp.int32),
        pltpu.VMEM((SB,), jnp.int32),
        pltpu.VMEM((16,), jnp.int32),
        pltpu.SemaphoreType.DMA,
        pltpu.SemaphoreType.DMA,
    ],
)
def _bucket(src_hbm, dst_hbm, recs_hbm, cnt_hbm,
            src0, dst0, src1, dst1, staging, cnt_v, sem0, sem1):
    b = _wid()
    lo = b * NPB
    hi = lo + NPB
    srcs = (src0, src1)
    dsts = (dst0, dst1)
    sems = (sem0, sem1)
    NCH = N_EDGES // CE

    def _issue(c, p):
        pltpu.async_copy(src_hbm.at[pl.ds(c * CE, CE)], srcs[p], sems[p])
        pltpu.async_copy(dst_hbm.at[pl.ds(c * CE, CE)], dsts[p], sems[p])

    def _wait(p):
        pltpu.make_async_copy(src_hbm.at[pl.ds(0, CE)], srcs[p],
                              sems[p]).wait()
        pltpu.make_async_copy(dst_hbm.at[pl.ds(0, CE)], dsts[p],
                              sems[p]).wait()

    def _compress(p, off):
        def _grp(g, off):
            s16 = srcs[p][pl.ds(g * 16, 16)]
            d16 = dsts[p][pl.ds(g * 16, 16)]
            m = jnp.logical_and(d16 >= lo, d16 < hi)
            rec = jnp.bitwise_or(lax.shift_left(s16, DSHIFT), d16 - lo)
            cs = plsc.cumsum(m.astype(jnp.int32))
            offv = jnp.full((16,), off, jnp.int32)
            plsc.store_scatter(staging, [offv + cs - 1], rec, mask=m)
            return off + cs[15]
        return lax.fori_loop(0, CE // 16, _grp, off)

    def _maybe_flush(carry):
        off, gofs = carry

        def _flush(c):
            off, gofs = c
            pltpu.sync_copy(
                staging.at[pl.ds(0, FLUSH)],
                recs_hbm.at[pl.ds(pl.multiple_of(b * CAP + gofs, 16), FLUSH)])
            nrem = off - FLUSH

            def _mv(i, _):
                staging[pl.ds(i * 16, 16)] = \
                    staging[pl.ds(FLUSH + i * 16, 16)]
                return 0
            lax.fori_loop(0, lax.div(nrem + 15, jnp.int32(16)), _mv, 0)
            return off - FLUSH, gofs + FLUSH

        return lax.cond(off >= FLUSH, _flush, lambda c: c, (off, gofs))

    _issue(0, 0)

    def _pair2(t, carry):
        c = 2 * t

        @pl.when(c + 1 < NCH)
        def _():
            _issue(c + 1, 1)

        _wait(0)
        carry = _maybe_flush((_compress(0, carry[0]), carry[1]))

        @pl.when(c + 2 < NCH)
        def _():
            _issue(c + 2, 0)

        _wait(1)
        carry = _maybe_flush((_compress(1, carry[0]), carry[1]))
        return carry

    off, gofs = lax.fori_loop(0, NCH // 2, _pair2,
                              (jnp.int32(0), jnp.int32(0)))

    dumpv = jnp.full((16,), DUMP, jnp.int32)
    staging[pl.ds(off, 16)] = dumpv
    staging[pl.ds(off + 16, 16)] = dumpv
    staging[pl.ds(off + 32, 16)] = dumpv
    staging[pl.ds(off + 48, 16)] = dumpv
    pltpu.sync_copy(
        staging, recs_hbm.at[pl.ds(pl.multiple_of(b * CAP + gofs, 16), SB)])
    cnt_v[...] = jnp.full((16,), gofs + off, jnp.int32)
    pltpu.sync_copy(cnt_v,
                    cnt_hbm.at[pl.ds(pl.multiple_of(b * 16, 16), 16)])


def _make_agg(d_feat):
    kv = d_feat // 16

    @functools.partial(
        pl.kernel,
        out_type=jax.ShapeDtypeStruct((NPAD, d_feat), jnp.float32),
        mesh=_sc_mesh,
        compiler_params=_sc_params,
        scratch_types=[
            pltpu.VMEM((NPB + 1, d_feat), jnp.float32),
            pltpu.VMEM((K, d_feat), jnp.float32),
            pltpu.VMEM((K, d_feat), jnp.float32),
            pltpu.VMEM((RB,), jnp.int32),
            pltpu.VMEM((K,), jnp.int32),
            pltpu.VMEM((K,), jnp.int32),
            pltpu.VMEM((16,), jnp.int32),
            pltpu.SemaphoreType.DMA,
            pltpu.SemaphoreType.DMA,
        ],
    )
    def _agg(x_hbm, recs_hbm, cnt_hbm, out_hbm,
             acc, rows0, rows1, rec_b, idx0, idx1, cnt_v, sem0, sem1):
        b = _wid()
        ninf = jnp.full((16,), NEG_INF, jnp.float32)
        rowss = (rows0, rows1)
        idxs = (idx0, idx1)
        sems = (sem0, sem1)

        def _init(n, _):
            for k in range(kv):
                acc[n, pl.ds(k * 16, 16)] = ninf
            return 0
        lax.fori_loop(0, NPB + 1, _init, 0)

        pltpu.sync_copy(cnt_hbm.at[pl.ds(pl.multiple_of(b * 16, 16), 16)],
                        cnt_v)
        cntb = cnt_v[...][0]
        nch = lax.div(cntb + (K - 1), jnp.int32(K))

        def _issue(ch, p):
            for g in range(K // 16):
                idxs[p][pl.ds(g * 16, 16)] = lax.shift_right_logical(
                    rec_b[pl.ds(pl.multiple_of(ch * K + g * 16, 16), 16)],
                    DSHIFT)
            pltpu.async_copy(x_hbm.at[idxs[p]], rowss[p], sems[p])

        def _finish(ch, p):
            pltpu.make_async_copy(x_hbm.at[idxs[p]], rowss[p], sems[p]).wait()

            def _grp(g, _):
                grp = rec_b[pl.ds(pl.multiple_of(ch * K + g * 16, 16), 16)]
                for l in range(16):
                    d = jnp.bitwise_and(grp[l], DMASK)
                    e = g * 16 + l
                    for k0 in range(0, kv, 8):
                        ks = range(k0, min(k0 + 8, kv))
                        curs = [acc[d, pl.ds(k * 16, 16)] for k in ks]
                        rs = [rowss[p][e, pl.ds(k * 16, 16)] for k in ks]
                        for i, k in enumerate(ks):
                            acc[d, pl.ds(k * 16, 16)] = jnp.maximum(
                                curs[i], rs[i])
                return 0
            lax.fori_loop(0, K // 16, _grp, 0)

        nblk = lax.div(cntb + (RB - 1), jnp.int32(RB))

        def _blk(blk, _):
            pltpu.sync_copy(
                recs_hbm.at[pl.ds(pl.multiple_of(b * CAP + blk * RB, 64), RB)],
                rec_b)
            nch_here = jnp.minimum(jnp.int32(CPB), nch - blk * CPB)

            @pl.when(nch_here > 0)
            def _():
                _issue(0, 0)

            def _pair(t, _):
                c0 = 2 * t
                c1 = c0 + 1
                c2 = c0 + 2

                @pl.when(c1 < nch_here)
                def _():
                    _issue(c1, 1)

                _finish(c0, 0)

                @pl.when(c2 < nch_here)
                def _():
                    _issue(c2, 0)

                @pl.when(c1 < nch_here)
                def _():
                    _finish(c1, 1)
                return 0
            lax.fori_loop(0, lax.div(nch_here + 1, jnp.int32(2)), _pair, 0)
            return 0
        lax.fori_loop(0, nblk, _blk, 0)

        def _fix(n, _):
            for k in range(kv):
                v = acc[n, pl.ds(k * 16, 16)]
                acc[n, pl.ds(k * 16, 16)] = jnp.where(v == NEG_INF, 0.0, v)
            return 0
        lax.fori_loop(0, NPB, _fix, 0)
        pltpu.sync_copy(
            acc.at[pl.ds(0, NPB)],
            out_hbm.at[pl.ds(pl.multiple_of(b * NPB, 8), NPB)])

    return _agg


_agg128 = _make_agg(128)
_agg256 = _make_agg(256)


BN = 1000
_GRID = N_NODES // BN


def _row_spec(d):
    return pl.BlockSpec((BN, d), lambda i: (i, 0))


def _full_spec(shape):
    nd = len(shape)
    return pl.BlockSpec(shape, lambda i: (0,) * nd)


def _tc_call(fn, ins, out_widths):
    in_specs = []
    for a in ins:
        if a.shape[0] == N_NODES:
            in_specs.append(_row_spec(a.shape[1]))
        else:
            in_specs.append(_full_spec(a.shape))
    multi = len(out_widths) > 1
    out_shape = tuple(jax.ShapeDtypeStruct((N_NODES, w), jnp.float32)
                      for w in out_widths)
    out_specs = tuple(_row_spec(w) for w in out_widths)
    if not multi:
        out_shape, out_specs = out_shape[0], out_specs[0]
    return pl.pallas_call(
        fn, grid=(_GRID,), in_specs=in_specs,
        out_specs=out_specs, out_shape=out_shape)(*ins)


def _stage_a(agg_ref, x_ref, wl, bl, wr, h_ref):
    h_ref[...] = jax.nn.relu(
        jnp.dot(agg_ref[...], wl[...], preferred_element_type=jnp.float32)
        + bl[...]
        + jnp.dot(x_ref[...], wr[...], preferred_element_type=jnp.float32))


def _stage_b(agg_ref, h_ref, wl, bl, wr, u1_ref, u2_ref):
    t = jax.nn.relu(
        jnp.dot(agg_ref[...], wl[...], preferred_element_type=jnp.float32)
        + bl[...]
        + jnp.dot(h_ref[...], wr[...], preferred_element_type=jnp.float32))
    u1_ref[...] = t[:, :256]
    u2_ref[...] = t[:, 256:]


def _stage_c(a1_ref, a2_ref, u1_ref, u2_ref, wl, bl, wr, v_ref, rm_ref):
    agg = jnp.concatenate([a1_ref[...], a2_ref[...]], axis=1)
    u = jnp.concatenate([u1_ref[...], u2_ref[...]], axis=1)
    t = jax.nn.relu(
        jnp.dot(agg, wl[...], preferred_element_type=jnp.float32)
        + bl[...]
        + jnp.dot(u, wr[...], preferred_element_type=jnp.float32))
    v_ref[...] = t
    rm_ref[...] = jnp.concatenate(
        [t[:, :64], jnp.zeros((BN, 64), jnp.float32)], axis=1)


def _stage_d(aggrm_ref, v_ref, wl3, bl3, wr3, wrt, brt, wmd, bmd, wj, bj,
             o_ref):
    v = v_ref[...]
    rm_pre = (jnp.dot(aggrm_ref[...][:, :64], wl3[...],
                      preferred_element_type=jnp.float32)
              + bl3[...]
              + jnp.dot(v[:, :64], wr3[...],
                        preferred_element_type=jnp.float32))
    m = jnp.max(rm_pre, axis=1, keepdims=True)
    ex = jnp.exp(rm_pre - m)
    ls = rm_pre - m - jnp.log(jnp.sum(ex, axis=1, keepdims=True))
    rt = jnp.dot(v[:, 64:128], wrt[...],
                 preferred_element_type=jnp.float32) + brt[...]
    md = jnp.dot(v[:, 128:192], wmd[...],
                 preferred_element_type=jnp.float32) + bmd[...]
    jj = jnp.dot(v[:, 192:256], wj[...],
                 preferred_element_type=jnp.float32) + bj[...]
    o_ref[...] = jnp.concatenate(
        [ls, rt, md, jj, jnp.zeros((BN, 2), jnp.float32)], axis=1)


def _block_diag4(ws):
    out = jnp.zeros((512, 256), jnp.float32)
    for i, w in enumerate(ws):
        out = out.at[i * 128:(i + 1) * 128, i * 64:(i + 1) * 64].set(w)
    return out


def kernel(x, edge_index, params):
    p = params
    src = edge_index[0]
    dst = edge_index[1]
    recs, cnt = _bucket(src, dst)

    aggx = _agg128(x, recs, cnt)[:N_NODES]
    h = _tc_call(_stage_a,
                 (aggx, x, p["shared"]["Wl"], p["shared"]["bl"].reshape(1, -1),
                  p["shared"]["Wr"]), (256,))

    aggh = _agg256(h, recs, cnt)[:N_NODES]
    wl_cat = jnp.concatenate(
        [p[k]["Wl"] for k in ("rm1", "rt1", "md1", "j1")], axis=1)
    bl_cat = jnp.concatenate(
        [p[k]["bl"] for k in ("rm1", "rt1", "md1", "j1")]).reshape(1, -1)
    wr_cat = jnp.concatenate(
        [p[k]["Wr"] for k in ("rm1", "rt1", "md1", "j1")], axis=1)
    u1, u2 = _tc_call(_stage_b, (aggh, h, wl_cat, bl_cat, wr_cat), (256, 256))

    a1 = _agg256(u1, recs, cnt)[:N_NODES]
    a2 = _agg256(u2, recs, cnt)[:N_NODES]
    wl_bd = _block_diag4([p[k]["Wl"] for k in ("rm2", "rt2", "md2", "j2")])
    bl_bd = jnp.concatenate(
        [p[k]["bl"] for k in ("rm2", "rt2", "md2", "j2")]).reshape(1, -1)
    wr_bd = _block_diag4([p[k]["Wr"] for k in ("rm2", "rt2", "md2", "j2")])
    v, rm2 = _tc_call(_stage_c, (a1, a2, u1, u2, wl_bd, bl_bd, wr_bd),
                      (256, 128))

    aggrm = _agg128(rm2, recs, cnt)[:N_NODES]
    o = _tc_call(_stage_d,
                 (aggrm, v, p["rm3"]["Wl"], p["rm3"]["bl"].reshape(1, -1),
                  p["rm3"]["Wr"], p["rt3"]["W"], p["rt3"]["b"].reshape(1, -1),
                  p["md3"]["W"], p["md3"]["b"].reshape(1, -1),
                  p["j3"]["W"], p["j3"]["b"].reshape(1, -1)), (8,))

    rm = o[:, 0:2]
    rt = o[:, 2]
    md = o[:, 3]
    j = o[:, 4:6].reshape(-1)
    return (rm, rt, md, j)


# speedup vs baseline: 2.9073x; 1.0053x over previous
import functools

import jax
import jax.numpy as jnp
from jax import lax
from jax.experimental import pallas as pl
from jax.experimental.pallas import tpu as pltpu
from jax.experimental.pallas import tpu_sc as plsc

N_NODES = 10000
N_EDGES = 320000
NB = 32
NPB = 320
NPAD = NB * NPB
K = 64
CE = 2000
SB = 4224
FLUSH = 2048
RB = 8192
CPB = RB // K
CAP = 40 * RB
DSHIFT = 9
DMASK = 511
DUMP = NPB
NEG_INF = float("-inf")

_sc_mesh = plsc.VectorSubcoreMesh(
    core_axis_name="c", subcore_axis_name="s", num_cores=2, num_subcores=16)
_sc_params = pltpu.CompilerParams(needs_layout_passes=False)


def _wid():
    return lax.axis_index("s") * 2 + lax.axis_index("c")


@functools.partial(
    pl.kernel,
    out_type=(
        jax.ShapeDtypeStruct((NB * CAP,), jnp.int32),
        jax.ShapeDtypeStruct((NB * 16,), jnp.int32),
    ),
    mesh=_sc_mesh,
    compiler_params=_sc_params,
    scratch_types=[
        pltpu.VMEM((CE,), jnp.int32),
        pltpu.VMEM((CE,), jnp.int32),
        pltpu.VMEM((CE,), jnp.int32),
        pltpu.VMEM((CE,), jnp.int32),
        pltpu.VMEM((SB,), jnp.int32),
        pltpu.VMEM((16,), jnp.int32),
        pltpu.SemaphoreType.DMA,
        pltpu.SemaphoreType.DMA,
    ],
)
def _bucket(src_hbm, dst_hbm, recs_hbm, cnt_hbm,
            src0, dst0, src1, dst1, staging, cnt_v, sem0, sem1):
    b = _wid()
    lo = b * NPB
    hi = lo + NPB
    srcs = (src0, src1)
    dsts = (dst0, dst1)
    sems = (sem0, sem1)
    NCH = N_EDGES // CE

    def _issue(c, p):
        pltpu.async_copy(src_hbm.at[pl.ds(c * CE, CE)], srcs[p], sems[p])
        pltpu.async_copy(dst_hbm.at[pl.ds(c * CE, CE)], dsts[p], sems[p])

    def _wait(p):
        pltpu.make_async_copy(src_hbm.at[pl.ds(0, CE)], srcs[p],
                              sems[p]).wait()
        pltpu.make_async_copy(dst_hbm.at[pl.ds(0, CE)], dsts[p],
                              sems[p]).wait()

    def _compress(p, off):
        def _grp(g, off):
            s16 = srcs[p][pl.ds(g * 16, 16)]
            d16 = dsts[p][pl.ds(g * 16, 16)]
            m = jnp.logical_and(d16 >= lo, d16 < hi)
            rec = jnp.bitwise_or(lax.shift_left(s16, DSHIFT), d16 - lo)
            cs = plsc.cumsum(m.astype(jnp.int32))
            offv = jnp.full((16,), off, jnp.int32)
            plsc.store_scatter(staging, [offv + cs - 1], rec, mask=m)
            return off + cs[15]
        return lax.fori_loop(0, CE // 16, _grp, off)

    def _maybe_flush(carry):
        off, gofs = carry

        def _flush(c):
            off, gofs = c
            pltpu.sync_copy(
                staging.at[pl.ds(0, FLUSH)],
                recs_hbm.at[pl.ds(pl.multiple_of(b * CAP + gofs, 16), FLUSH)])
            nrem = off - FLUSH

            def _mv(i, _):
                staging[pl.ds(i * 16, 16)] = \
                    staging[pl.ds(FLUSH + i * 16, 16)]
                return 0
            lax.fori_loop(0, lax.div(nrem + 15, jnp.int32(16)), _mv, 0)
            return off - FLUSH, gofs + FLUSH

        return lax.cond(off >= FLUSH, _flush, lambda c: c, (off, gofs))

    _issue(0, 0)

    def _pair2(t, carry):
        c = 2 * t

        @pl.when(c + 1 < NCH)
        def _():
            _issue(c + 1, 1)

        _wait(0)
        carry = _maybe_flush((_compress(0, carry[0]), carry[1]))

        @pl.when(c + 2 < NCH)
        def _():
            _issue(c + 2, 0)

        _wait(1)
        carry = _maybe_flush((_compress(1, carry[0]), carry[1]))
        return carry

    off, gofs = lax.fori_loop(0, NCH // 2, _pair2,
                              (jnp.int32(0), jnp.int32(0)))

    dumpv = jnp.full((16,), DUMP, jnp.int32)
    staging[pl.ds(off, 16)] = dumpv
    staging[pl.ds(off + 16, 16)] = dumpv
    staging[pl.ds(off + 32, 16)] = dumpv
    staging[pl.ds(off + 48, 16)] = dumpv
    pltpu.sync_copy(
        staging, recs_hbm.at[pl.ds(pl.multiple_of(b * CAP + gofs, 16), SB)])
    cnt_v[...] = jnp.full((16,), gofs + off, jnp.int32)
    pltpu.sync_copy(cnt_v,
                    cnt_hbm.at[pl.ds(pl.multiple_of(b * 16, 16), 16)])


def _make_agg(d_feat):
    kv = d_feat // 16

    @functools.partial(
        pl.kernel,
        out_type=jax.ShapeDtypeStruct((NPAD, d_feat), jnp.float32),
        mesh=_sc_mesh,
        compiler_params=_sc_params,
        scratch_types=[
            pltpu.VMEM((NPB + 1, d_feat), jnp.float32),
            pltpu.VMEM((K, d_feat), jnp.float32),
            pltpu.VMEM((K, d_feat), jnp.float32),
            pltpu.VMEM((RB,), jnp.int32),
            pltpu.VMEM((K,), jnp.int32),
            pltpu.VMEM((K,), jnp.int32),
            pltpu.VMEM((16,), jnp.int32),
            pltpu.SemaphoreType.DMA,
            pltpu.SemaphoreType.DMA,
        ],
    )
    def _agg(x_hbm, recs_hbm, cnt_hbm, out_hbm,
             acc, rows0, rows1, rec_b, idx0, idx1, cnt_v, sem0, sem1):
        b = _wid()
        ninf = jnp.full((16,), NEG_INF, jnp.float32)
        rowss = (rows0, rows1)
        idxs = (idx0, idx1)
        sems = (sem0, sem1)

        def _init(n, _):
            for k in range(kv):
                acc[n, pl.ds(k * 16, 16)] = ninf
            return 0
        lax.fori_loop(0, NPB + 1, _init, 0)

        pltpu.sync_copy(cnt_hbm.at[pl.ds(pl.multiple_of(b * 16, 16), 16)],
                        cnt_v)
        cntb = cnt_v[...][0]
        nch = lax.div(cntb + (K - 1), jnp.int32(K))

        def _issue(ch, p):
            for g in range(K // 16):
                idxs[p][pl.ds(g * 16, 16)] = lax.shift_right_logical(
                    rec_b[pl.ds(pl.multiple_of(ch * K + g * 16, 16), 16)],
                    DSHIFT)
            pltpu.async_copy(x_hbm.at[idxs[p]], rowss[p], sems[p])

        def _finish(ch, p):
            pltpu.make_async_copy(x_hbm.at[idxs[p]], rowss[p], sems[p]).wait()

            def _grp(g, _):
                grp = rec_b[pl.ds(pl.multiple_of(ch * K + g * 16, 16), 16)]
                for l in range(16):
                    d = jnp.bitwise_and(grp[l], DMASK)
                    e = g * 16 + l
                    for k0 in range(0, kv, 16):
                        ks = range(k0, min(k0 + 16, kv))
                        curs = [acc[d, pl.ds(k * 16, 16)] for k in ks]
                        rs = [rowss[p][e, pl.ds(k * 16, 16)] for k in ks]
                        for i, k in enumerate(ks):
                            acc[d, pl.ds(k * 16, 16)] = jnp.maximum(
                                curs[i], rs[i])
                return 0
            lax.fori_loop(0, K // 16, _grp, 0)

        nblk = lax.div(cntb + (RB - 1), jnp.int32(RB))

        def _blk(blk, _):
            pltpu.sync_copy(
                recs_hbm.at[pl.ds(pl.multiple_of(b * CAP + blk * RB, 64), RB)],
                rec_b)
            nch_here = jnp.minimum(jnp.int32(CPB), nch - blk * CPB)

            @pl.when(nch_here > 0)
            def _():
                _issue(0, 0)

            def _pair(t, _):
                c0 = 2 * t
                c1 = c0 + 1
                c2 = c0 + 2

                @pl.when(c1 < nch_here)
                def _():
                    _issue(c1, 1)

                _finish(c0, 0)

                @pl.when(c2 < nch_here)
                def _():
                    _issue(c2, 0)

                @pl.when(c1 < nch_here)
                def _():
                    _finish(c1, 1)
                return 0
            lax.fori_loop(0, lax.div(nch_here + 1, jnp.int32(2)), _pair, 0)
            return 0
        lax.fori_loop(0, nblk, _blk, 0)

        def _fix(n, _):
            for k in range(kv):
                v = acc[n, pl.ds(k * 16, 16)]
                acc[n, pl.ds(k * 16, 16)] = jnp.where(v == NEG_INF, 0.0, v)
            return 0
        lax.fori_loop(0, NPB, _fix, 0)
        pltpu.sync_copy(
            acc.at[pl.ds(0, NPB)],
            out_hbm.at[pl.ds(pl.multiple_of(b * NPB, 8), NPB)])

    return _agg


_agg128 = _make_agg(128)
_agg256 = _make_agg(256)


BN = 1000
_GRID = N_NODES // BN


def _row_spec(d):
    return pl.BlockSpec((BN, d), lambda i: (i, 0))


def _full_spec(shape):
    nd = len(shape)
    return pl.BlockSpec(shape, lambda i: (0,) * nd)


def _tc_call(fn, ins, out_widths):
    in_specs = []
    for a in ins:
        if a.shape[0] == N_NODES:
            in_specs.append(_row_spec(a.shape[1]))
        else:
            in_specs.append(_full_spec(a.shape))
    multi = len(out_widths) > 1
    out_shape = tuple(jax.ShapeDtypeStruct((N_NODES, w), jnp.float32)
                      for w in out_widths)
    out_specs = tuple(_row_spec(w) for w in out_widths)
    if not multi:
        out_shape, out_specs = out_shape[0], out_specs[0]
    return pl.pallas_call(
        fn, grid=(_GRID,), in_specs=in_specs,
        out_specs=out_specs, out_shape=out_shape)(*ins)


def _stage_a(agg_ref, x_ref, wl, bl, wr, h_ref):
    h_ref[...] = jax.nn.relu(
        jnp.dot(agg_ref[...], wl[...], preferred_element_type=jnp.float32)
        + bl[...]
        + jnp.dot(x_ref[...], wr[...], preferred_element_type=jnp.float32))


def _stage_b(agg_ref, h_ref, wl, bl, wr, u1_ref, u2_ref):
    t = jax.nn.relu(
        jnp.dot(agg_ref[...], wl[...], preferred_element_type=jnp.float32)
        + bl[...]
        + jnp.dot(h_ref[...], wr[...], preferred_element_type=jnp.float32))
    u1_ref[...] = t[:, :256]
    u2_ref[...] = t[:, 256:]


def _stage_c(a1_ref, a2_ref, u1_ref, u2_ref, wl, bl, wr, v_ref, rm_ref):
    agg = jnp.concatenate([a1_ref[...], a2_ref[...]], axis=1)
    u = jnp.concatenate([u1_ref[...], u2_ref[...]], axis=1)
    t = jax.nn.relu(
        jnp.dot(agg, wl[...], preferred_element_type=jnp.float32)
        + bl[...]
        + jnp.dot(u, wr[...], preferred_element_type=jnp.float32))
    v_ref[...] = t
    rm_ref[...] = jnp.concatenate(
        [t[:, :64], jnp.zeros((BN, 64), jnp.float32)], axis=1)


def _stage_d(aggrm_ref, v_ref, wl3, bl3, wr3, wrt, brt, wmd, bmd, wj, bj,
             o_ref):
    v = v_ref[...]
    rm_pre = (jnp.dot(aggrm_ref[...][:, :64], wl3[...],
                      preferred_element_type=jnp.float32)
              + bl3[...]
              + jnp.dot(v[:, :64], wr3[...],
                        preferred_element_type=jnp.float32))
    m = jnp.max(rm_pre, axis=1, keepdims=True)
    ex = jnp.exp(rm_pre - m)
    ls = rm_pre - m - jnp.log(jnp.sum(ex, axis=1, keepdims=True))
    rt = jnp.dot(v[:, 64:128], wrt[...],
                 preferred_element_type=jnp.float32) + brt[...]
    md = jnp.dot(v[:, 128:192], wmd[...],
                 preferred_element_type=jnp.float32) + bmd[...]
    jj = jnp.dot(v[:, 192:256], wj[...],
                 preferred_element_type=jnp.float32) + bj[...]
    o_ref[...] = jnp.concatenate(
        [ls, rt, md, jj, jnp.zeros((BN, 2), jnp.float32)], axis=1)


def _block_diag4(ws):
    out = jnp.zeros((512, 256), jnp.float32)
    for i, w in enumerate(ws):
        out = out.at[i * 128:(i + 1) * 128, i * 64:(i + 1) * 64].set(w)
    return out


def kernel(x, edge_index, params):
    p = params
    src = edge_index[0]
    dst = edge_index[1]
    recs, cnt = _bucket(src, dst)

    aggx = _agg128(x, recs, cnt)[:N_NODES]
    h = _tc_call(_stage_a,
                 (aggx, x, p["shared"]["Wl"], p["shared"]["bl"].reshape(1, -1),
                  p["shared"]["Wr"]), (256,))

    aggh = _agg256(h, recs, cnt)[:N_NODES]
    wl_cat = jnp.concatenate(
        [p[k]["Wl"] for k in ("rm1", "rt1", "md1", "j1")], axis=1)
    bl_cat = jnp.concatenate(
        [p[k]["bl"] for k in ("rm1", "rt1", "md1", "j1")]).reshape(1, -1)
    wr_cat = jnp.concatenate(
        [p[k]["Wr"] for k in ("rm1", "rt1", "md1", "j1")], axis=1)
    u1, u2 = _tc_call(_stage_b, (aggh, h, wl_cat, bl_cat, wr_cat), (256, 256))

    a1 = _agg256(u1, recs, cnt)[:N_NODES]
    a2 = _agg256(u2, recs, cnt)[:N_NODES]
    wl_bd = _block_diag4([p[k]["Wl"] for k in ("rm2", "rt2", "md2", "j2")])
    bl_bd = jnp.concatenate(
        [p[k]["bl"] for k in ("rm2", "rt2", "md2", "j2")]).reshape(1, -1)
    wr_bd = _block_diag4([p[k]["Wr"] for k in ("rm2", "rt2", "md2", "j2")])
    v, rm2 = _tc_call(_stage_c, (a1, a2, u1, u2, wl_bd, bl_bd, wr_bd),
                      (256, 128))

    aggrm = _agg128(rm2, recs, cnt)[:N_NODES]
    o = _tc_call(_stage_d,
                 (aggrm, v, p["rm3"]["Wl"], p["rm3"]["bl"].reshape(1, -1),
                  p["rm3"]["Wr"], p["rt3"]["W"], p["rt3"]["b"].reshape(1, -1),
                  p["md3"]["W"], p["md3"]["b"].reshape(1, -1),
                  p["j3"]["W"], p["j3"]["b"].reshape(1, -1)), (8,))

    rm = o[:, 0:2]
    rt = o[:, 2]
    md = o[:, 3]
    j = o[:, 4:6].reshape(-1)
    return (rm, rt, md, j)
